# Initial kernel scaffold; baseline (speedup 1.0000x reference)
import jax
import jax.numpy as jnp
from jax.experimental import pallas as pl


def kernel(user_idx, item_idx, neg_item_idx, edge_uu, edge_ii0, edge_ii1, feat_user, feat_item, gat_u_W, gat_u_al, gat_u_ar, gat_i0_W, gat_i0_al, gat_i0_ar, gat_i1_W, gat_i1_al, gat_i1_ar, sem_u_W1, sem_u_b1, sem_u_W2, sem_i_W1, sem_i_b1, sem_i_W2, user_W, user_b, item_W, item_b, ln_g, ln_b):
    raise NotImplementedError("write your pallas kernel here")


# speedup vs baseline: 1.0542x; 1.0542x over previous
import functools
import jax
import jax.numpy as jnp
from jax.experimental import pallas as pl

N_NODES = 10000
D = 128


def _proj_body(x_ref, w_ref, al_ref, ar_ref, h_ref, elr_ref):
    h = jnp.dot(x_ref[...], w_ref[...], preferred_element_type=jnp.float32)
    h_ref[...] = h
    el = h @ al_ref[...].reshape(D, 1)
    er = h @ ar_ref[...].reshape(D, 1)
    elr_ref[...] = jnp.concatenate([el, er], axis=1)


def _project(x, w, al, ar):
    n = x.shape[0]
    return pl.pallas_call(
        _proj_body,
        out_shape=(
            jax.ShapeDtypeStruct((n, D), jnp.float32),
            jax.ShapeDtypeStruct((n, 2), jnp.float32),
        ),
    )(x, w, al.reshape(1, D), ar.reshape(1, D))


def _edge_softmax_agg(h, elr, edge_index):
    src = edge_index[0]
    dst = edge_index[1]
    el = elr[:, 0]
    er = elr[:, 1]
    e = jax.nn.leaky_relu(el[src] + er[dst], 0.2)
    n = h.shape[0]
    m = jax.ops.segment_max(e, dst, num_segments=n)
    ee = jnp.exp(e - m[dst])
    s = jax.ops.segment_sum(ee, dst, num_segments=n)
    alpha = ee / (s[dst] + 1e-9)
    return jax.ops.segment_sum(alpha[:, None] * h[src], dst, num_segments=n)


def _elu(x):
    return jnp.where(x > 0, x, jnp.exp(jnp.minimum(x, 0.0)) - 1.0)


def _fuse_body(aggu_ref, aggi0_ref, aggi1_ref,
               semw1_ref, semb1_ref, semw2_ref,
               userw_ref, userb_ref, itemw_ref, itemb_ref,
               lng_ref, lnb_ref,
               uemb_ref, iemb_ref):
    n = aggu_ref.shape[0]
    ue = _elu(aggu_ref[...])
    ue = jax.nn.relu(jnp.dot(ue, userw_ref[...],
                             preferred_element_type=jnp.float32)
                     + userb_ref[...])
    mu = ue.mean(-1, keepdims=True)
    var = ((ue - mu) ** 2).mean(-1, keepdims=True)
    uemb_ref[...] = (ue - mu) / jnp.sqrt(var + 1e-5) * lng_ref[...] + lnb_ref[...]

    z0 = _elu(aggi0_ref[...])
    z1 = _elu(aggi1_ref[...])
    w1 = semw1_ref[...]
    b1 = semb1_ref[...]
    w2 = semw2_ref[...]
    q0 = jnp.tanh(jnp.dot(z0, w1, preferred_element_type=jnp.float32) + b1)
    q1 = jnp.tanh(jnp.dot(z1, w1, preferred_element_type=jnp.float32) + b1)
    s0 = jnp.sum(jnp.sum(q0, axis=0) * w2[0]) / n
    s1 = jnp.sum(jnp.sum(q1, axis=0) * w2[0]) / n
    mx = jnp.maximum(s0, s1)
    e0 = jnp.exp(s0 - mx)
    e1 = jnp.exp(s1 - mx)
    b0 = e0 / (e0 + e1)
    bb1 = e1 / (e0 + e1)
    ie = b0 * z0 + bb1 * z1
    ie = jax.nn.relu(jnp.dot(ie, itemw_ref[...],
                             preferred_element_type=jnp.float32)
                     + itemb_ref[...])
    mi = ie.mean(-1, keepdims=True)
    vi = ((ie - mi) ** 2).mean(-1, keepdims=True)
    iemb_ref[...] = (ie - mi) / jnp.sqrt(vi + 1e-5) * lng_ref[...] + lnb_ref[...]


def _fuse(agg_u, agg_i0, agg_i1, sem_i_W1, sem_i_b1, sem_i_W2,
          user_W, user_b, item_W, item_b, ln_g, ln_b):
    n = agg_u.shape[0]
    return pl.pallas_call(
        _fuse_body,
        out_shape=(
            jax.ShapeDtypeStruct((n, D), jnp.float32),
            jax.ShapeDtypeStruct((n, D), jnp.float32),
        ),
    )(agg_u, agg_i0, agg_i1,
      sem_i_W1, sem_i_b1.reshape(1, D), sem_i_W2.reshape(1, D),
      user_W, user_b.reshape(1, D), item_W, item_b.reshape(1, D),
      ln_g.reshape(1, D), ln_b.reshape(1, D))


@jax.jit
def kernel(user_idx, item_idx, neg_item_idx, edge_uu, edge_ii0, edge_ii1,
           feat_user, feat_item, gat_u_W, gat_u_al, gat_u_ar,
           gat_i0_W, gat_i0_al, gat_i0_ar, gat_i1_W, gat_i1_al, gat_i1_ar,
           sem_u_W1, sem_u_b1, sem_u_W2, sem_i_W1, sem_i_b1, sem_i_W2,
           user_W, user_b, item_W, item_b, ln_g, ln_b):
    hu, elru = _project(feat_user, gat_u_W, gat_u_al, gat_u_ar)
    hi0, elri0 = _project(feat_item, gat_i0_W, gat_i0_al, gat_i0_ar)
    hi1, elri1 = _project(feat_item, gat_i1_W, gat_i1_al, gat_i1_ar)

    agg_u = _edge_softmax_agg(hu, elru, edge_uu)
    agg_i0 = _edge_softmax_agg(hi0, elri0, edge_ii0)
    agg_i1 = _edge_softmax_agg(hi1, elri1, edge_ii1)

    user_emb, item_emb = _fuse(agg_u, agg_i0, agg_i1,
                               sem_i_W1, sem_i_b1, sem_i_W2,
                               user_W, user_b, item_W, item_b, ln_g, ln_b)

    return (user_emb[user_idx], item_emb[item_idx], item_emb[neg_item_idx])


# speedup vs baseline: 13.6233x; 12.9232x over previous
import functools
import jax
import jax.numpy as jnp
from jax import lax
from jax.experimental import pallas as pl
from jax.experimental.pallas import tpu as pltpu
from jax.experimental.pallas import tpu_sc as plsc

N_NODES = 10000
N_PAD = 10112
D = 128
E = 320000
NW = 32
CHUNKS = 125
K = 80
HALF = 5120
NH = 5248


def _proj_body(x_ref, w_ref, al_ref, ar_ref, h_ref, elr_ref):
    h = jnp.dot(x_ref[...], w_ref[...], preferred_element_type=jnp.float32)
    h_ref[...] = h
    el = h @ al_ref[...].reshape(D, 1)
    er = h @ ar_ref[...].reshape(D, 1)
    elmax = jnp.max(el)
    mub = elmax + er
    mub = jnp.where(mub > 0, mub, 0.2 * mub)
    elr_ref[...] = jnp.concatenate([el, er, mub, el], axis=1)


def _project(x, w, al, ar):
    n = x.shape[0]
    return pl.pallas_call(
        _proj_body,
        out_shape=(
            jax.ShapeDtypeStruct((n, D), jnp.float32),
            jax.ShapeDtypeStruct((n, 4), jnp.float32),
        ),
    )(x, w, al.reshape(1, D), ar.reshape(1, D))


def _lrelu16(x):
    return jnp.where(x > 0, x, 0.2 * x)


def _edge_logits_body(src_hbm, dst_hbm, el_hbm, er_hbm, mub_hbm,
                      ee_hbm, s_hbm,
                      el_v, er_v, mub_v, src_v, dst_v, ee_v, s_v):
    cid = lax.axis_index("c")
    sid = lax.axis_index("s")
    wid = sid * 2 + cid
    pltpu.sync_copy(el_hbm, el_v)
    pltpu.sync_copy(er_hbm, er_v)
    pltpu.sync_copy(mub_hbm, mub_v)
    pltpu.sync_copy(src_hbm.at[wid], src_v)
    pltpu.sync_copy(dst_hbm.at[wid], dst_v)

    zeros16 = jnp.zeros((16,), jnp.float32)

    def zero_body(j, _):
        s_v[pl.ds(j * 16, 16)] = zeros16
        return _
    lax.fori_loop(0, N_PAD // 16, zero_body, None)

    def chunk_body(ci, _):
        for q in range(K // 16):
            sidx = src_v[ci, pl.ds(q * 16, 16)]
            didx = dst_v[ci, pl.ds(q * 16, 16)]
            els = plsc.load_gather(el_v, [sidx])
            erd = plsc.load_gather(er_v, [didx])
            mubd = plsc.load_gather(mub_v, [didx])
            e = _lrelu16(els + erd)
            ee = jnp.exp(e - mubd)
            ee_v[ci, pl.ds(q * 16, 16)] = ee
            plsc.addupdate_scatter(s_v, [didx], ee)
        return _
    lax.fori_loop(0, CHUNKS, chunk_body, None)

    pltpu.sync_copy(ee_v, ee_hbm.at[wid])
    pltpu.sync_copy(s_v, s_hbm.at[wid])


def _sc_edge_logits(src3, dst3, el, er, mub):
    mesh = plsc.VectorSubcoreMesh(core_axis_name="c", subcore_axis_name="s")
    f = functools.partial(
        pl.kernel,
        mesh=mesh,
        out_type=(
            jax.ShapeDtypeStruct((NW, CHUNKS, K), jnp.float32),
            jax.ShapeDtypeStruct((NW, N_PAD), jnp.float32),
        ),
        scratch_types=[
            pltpu.VMEM((N_NODES,), jnp.float32),
            pltpu.VMEM((N_NODES,), jnp.float32),
            pltpu.VMEM((N_NODES,), jnp.float32),
            pltpu.VMEM((CHUNKS, K), jnp.int32),
            pltpu.VMEM((CHUNKS, K), jnp.int32),
            pltpu.VMEM((CHUNKS, K), jnp.float32),
            pltpu.VMEM((N_PAD,), jnp.float32),
        ],
        compiler_params=pltpu.CompilerParams(needs_layout_passes=False),
    )(_edge_logits_body)
    return f(src3, dst3, el, er, mub)


def _aggregate_body(base, src_hbm, dst_hbm, ee_hbm, s_hbm, h_hbm, z2_hbm,
                    out_hbm,
                    s_v, src_v, dst_v, ee_v, rows_v, shared_out, sem):
    cid = lax.axis_index("c")
    sid = lax.axis_index("s")
    wid = sid * 2 + cid
    pltpu.sync_copy(s_hbm, s_v)
    pltpu.sync_copy(src_hbm.at[wid], src_v)
    pltpu.sync_copy(dst_hbm.at[wid], dst_v)
    pltpu.sync_copy(ee_hbm.at[wid], ee_v)

    def alpha_body(ci, _):
        for q in range(K // 16):
            didx = dst_v[ci, pl.ds(q * 16, 16)]
            ee = ee_v[ci, pl.ds(q * 16, 16)]
            sd = plsc.load_gather(s_v, [didx])
            ee_v[ci, pl.ds(q * 16, 16)] = ee / sd
        return _
    lax.fori_loop(0, CHUNKS, alpha_body, None)

    def remap_body(ci, _):
        for q in range(K // 16):
            d16 = dst_v[ci, pl.ds(q * 16, 16)]
            loc = d16 - base
            ok = (loc >= 0) & (loc < HALF)
            dst_v[ci, pl.ds(q * 16, 16)] = jnp.where(ok, loc, HALF)
        return _
    lax.fori_loop(0, CHUNKS, remap_body, None)

    pltpu.sync_copy(z2_hbm.at[pl.ds(sid * 328, 328)],
                    shared_out.at[pl.ds(sid * 328, 328)])
    plsc.subcore_barrier()

    def chunk_body(ci, _):
        pltpu.async_copy(h_hbm.at[src_v.at[ci]], rows_v, sem).wait()

        def row_body(j, _2):
            a16 = plsc.load_gather(
                ee_v, [jnp.full((16,), ci, jnp.int32),
                       jnp.full((16,), j, jnp.int32)])
            for qq in range(D // 16):
                rows_v[j, pl.ds(qq * 16, 16)] = (
                    rows_v[j, pl.ds(qq * 16, 16)] * a16)
            return _2
        lax.fori_loop(0, K, row_body, None)

        pltpu.sync_copy(rows_v, shared_out.at[dst_v.at[ci]], add=True)
        return _
    lax.fori_loop(0, CHUNKS, chunk_body, None)

    plsc.subcore_barrier()
    pltpu.sync_copy(shared_out.at[pl.ds(sid * 328, 328)],
                    out_hbm.at[cid, pl.ds(sid * 328, 328)])


def _sc_aggregate(src3, dst3, ee3, s, h, z2, base):
    mesh = plsc.VectorSubcoreMesh(core_axis_name="c", subcore_axis_name="s")
    f = functools.partial(
        pl.kernel,
        mesh=mesh,
        out_type=jax.ShapeDtypeStruct((2, NH, D), jnp.float32),
        scratch_types=[
            pltpu.VMEM((N_PAD,), jnp.float32),
            pltpu.VMEM((CHUNKS, K), jnp.int32),
            pltpu.VMEM((CHUNKS, K), jnp.int32),
            pltpu.VMEM((CHUNKS, K), jnp.float32),
            pltpu.VMEM((K, D), jnp.float32),
            pltpu.VMEM_SHARED((NH, D), jnp.float32),
            pltpu.SemaphoreType.DMA,
        ],
        compiler_params=pltpu.CompilerParams(needs_layout_passes=False),
    )(functools.partial(_aggregate_body, base))
    return f(src3, dst3, ee3, s, h, z2)


def _gat_edge_phase(h, elr, edge_index, z2):
    src3 = edge_index[0].reshape(NW, CHUNKS, K)
    dst3 = edge_index[1].reshape(NW, CHUNKS, K)
    el = elr[:, 0]
    er = elr[:, 1]
    mub = elr[:, 2]
    ee3, s32 = _sc_edge_logits(src3, dst3, el, er, mub)
    s = s32.sum(0) + 1e-9
    outp0 = _sc_aggregate(src3, dst3, ee3, s, h, z2, 0)
    outp1 = _sc_aggregate(src3, dst3, ee3, s, h, z2, HALF)
    return jnp.concatenate([outp0[:, :HALF], outp1[:, :N_NODES - HALF]],
                           axis=1)


def _elu(x):
    return jnp.where(x > 0, x, jnp.exp(jnp.minimum(x, 0.0)) - 1.0)


def _fuse_body(aggu_ref, aggi0_ref, aggi1_ref,
               semw1_ref, semb1_ref, semw2_ref,
               userw_ref, userb_ref, itemw_ref, itemb_ref,
               lng_ref, lnb_ref,
               uemb_ref, iemb_ref):
    n = N_NODES
    aggu = aggu_ref[0, :n, :] + aggu_ref[1, :n, :]
    aggi0 = aggi0_ref[0, :n, :] + aggi0_ref[1, :n, :]
    aggi1 = aggi1_ref[0, :n, :] + aggi1_ref[1, :n, :]
    ue = _elu(aggu)
    ue = jax.nn.relu(jnp.dot(ue, userw_ref[...],
                             preferred_element_type=jnp.float32)
                     + userb_ref[...])
    mu = ue.mean(-1, keepdims=True)
    var = ((ue - mu) ** 2).mean(-1, keepdims=True)
    uemb_ref[...] = (ue - mu) / jnp.sqrt(var + 1e-5) * lng_ref[...] + lnb_ref[...]

    z0 = _elu(aggi0)
    z1 = _elu(aggi1)
    w1 = semw1_ref[...]
    b1 = semb1_ref[...]
    w2 = semw2_ref[...]
    q0 = jnp.tanh(jnp.dot(z0, w1, preferred_element_type=jnp.float32) + b1)
    q1 = jnp.tanh(jnp.dot(z1, w1, preferred_element_type=jnp.float32) + b1)
    s0 = jnp.sum(jnp.sum(q0, axis=0) * w2[0]) / n
    s1 = jnp.sum(jnp.sum(q1, axis=0) * w2[0]) / n
    mx = jnp.maximum(s0, s1)
    e0 = jnp.exp(s0 - mx)
    e1 = jnp.exp(s1 - mx)
    b0 = e0 / (e0 + e1)
    bb1 = e1 / (e0 + e1)
    ie = b0 * z0 + bb1 * z1
    ie = jax.nn.relu(jnp.dot(ie, itemw_ref[...],
                             preferred_element_type=jnp.float32)
                     + itemb_ref[...])
    mi = ie.mean(-1, keepdims=True)
    vi = ((ie - mi) ** 2).mean(-1, keepdims=True)
    iemb_ref[...] = (ie - mi) / jnp.sqrt(vi + 1e-5) * lng_ref[...] + lnb_ref[...]


def _fuse(agg_u, agg_i0, agg_i1, sem_i_W1, sem_i_b1, sem_i_W2,
          user_W, user_b, item_W, item_b, ln_g, ln_b):
    n = N_NODES
    return pl.pallas_call(
        _fuse_body,
        out_shape=(
            jax.ShapeDtypeStruct((n, D), jnp.float32),
            jax.ShapeDtypeStruct((n, D), jnp.float32),
        ),
    )(agg_u, agg_i0, agg_i1,
      sem_i_W1, sem_i_b1.reshape(1, D), sem_i_W2.reshape(1, D),
      user_W, user_b.reshape(1, D), item_W, item_b.reshape(1, D),
      ln_g.reshape(1, D), ln_b.reshape(1, D))


@jax.jit
def kernel(user_idx, item_idx, neg_item_idx, edge_uu, edge_ii0, edge_ii1,
           feat_user, feat_item, gat_u_W, gat_u_al, gat_u_ar,
           gat_i0_W, gat_i0_al, gat_i0_ar, gat_i1_W, gat_i1_al, gat_i1_ar,
           sem_u_W1, sem_u_b1, sem_u_W2, sem_i_W1, sem_i_b1, sem_i_W2,
           user_W, user_b, item_W, item_b, ln_g, ln_b):
    hu, elru = _project(feat_user, gat_u_W, gat_u_al, gat_u_ar)
    hi0, elri0 = _project(feat_item, gat_i0_W, gat_i0_al, gat_i0_ar)
    hi1, elri1 = _project(feat_item, gat_i1_W, gat_i1_al, gat_i1_ar)

    z2 = jnp.zeros((NH, D), jnp.float32)
    agg_u = _gat_edge_phase(hu, elru, edge_uu, z2)
    agg_i0 = _gat_edge_phase(hi0, elri0, edge_ii0, z2)
    agg_i1 = _gat_edge_phase(hi1, elri1, edge_ii1, z2)

    user_emb, item_emb = _fuse(agg_u, agg_i0, agg_i1,
                               sem_i_W1, sem_i_b1, sem_i_W2,
                               user_W, user_b, item_W, item_b, ln_g, ln_b)

    return (user_emb[user_idx], item_emb[item_idx], item_emb[neg_item_idx])
